# baseline (speedup 1.0000x reference)
import functools

import jax
import jax.numpy as jnp
from jax import lax
from jax.experimental import pallas as pl
from jax.experimental.pallas import tpu as pltpu
from jax.experimental.pallas import tpu_sc as plsc

_N = 10000
_E = 320000
_D = 128
_DH = _D // 2
_NC = 2
_NS = 16
_NW = _NC * _NS
_C = 128
_EP = 327680
_ROWS = _EP // _C
_RPT = _ROWS // _NS
_GRP = 16
_NGRP = _RPT // _GRP
_NPAD = 10240
_WB = _NPAD // _NS
_BN = 1000


def _deg_body(dst_hbm, out_hbm, hist, idx, ones, zbuf):
    c = lax.axis_index("c")
    s = lax.axis_index("s")
    wid = s * _NC + c

    def fill_ones(i, carry):
        ones[pl.ds(i * 16, 16)] = jnp.ones((16,), jnp.float32)
        return carry

    lax.fori_loop(0, _C // 16, fill_ones, 0)

    @pl.when(s == 0)
    def _init():
        def fill_z(i, carry):
            zbuf[pl.ds(i * 16, 16)] = jnp.zeros((16,), jnp.float32)
            return carry

        lax.fori_loop(0, 128, fill_z, 0)
        for j in range(_NPAD // 2048):
            pltpu.sync_copy(zbuf, hist.at[pl.ds(j * 2048, 2048)])

    plsc.subcore_barrier()

    base_row = wid * (_ROWS // _NW)

    def group(g, carry):
        pltpu.sync_copy(dst_hbm.at[pl.ds(base_row + g * _GRP, _GRP), :], idx)
        for j in range(_GRP):
            pltpu.sync_copy(ones, hist.at[idx.at[j]], add=True)
        return carry

    lax.fori_loop(0, (_ROWS // _NW) // _GRP, group, 0)

    plsc.subcore_barrier()
    nb = s * _WB
    pltpu.sync_copy(hist.at[pl.ds(nb, _WB)], out_hbm.at[c, pl.ds(nb, _WB)])


def _agg_body(hsL_hbm, hsR_hbm, src_hbm, dst_hbm, out_hbm,
              table, acc, sidx, didx, rows, semg, sems):
    c = lax.axis_index("c")
    s = lax.axis_index("s")
    nb = s * _WB

    @pl.when(c == 0)
    def _stage0():
        pltpu.sync_copy(hsL_hbm.at[pl.ds(nb, _WB), :], table.at[pl.ds(nb, _WB), :])
        pltpu.sync_copy(hsL_hbm.at[pl.ds(nb, _WB), :], acc.at[pl.ds(nb, _WB), :])

    @pl.when(c == 1)
    def _stage1():
        pltpu.sync_copy(hsR_hbm.at[pl.ds(nb, _WB), :], table.at[pl.ds(nb, _WB), :])
        pltpu.sync_copy(hsR_hbm.at[pl.ds(nb, _WB), :], acc.at[pl.ds(nb, _WB), :])

    plsc.subcore_barrier()

    base_row = s * _RPT

    def group(g, carry):
        pltpu.sync_copy(src_hbm.at[pl.ds(base_row + g * _GRP, _GRP), :], sidx)
        pltpu.sync_copy(dst_hbm.at[pl.ds(base_row + g * _GRP, _GRP), :], didx)
        gd = [None] * _GRP
        sd = [None] * _GRP
        gd[0] = pltpu.async_copy(table.at[sidx.at[0]], rows.at[0], semg)
        for j in range(_GRP):
            gd[j].wait()
            sd[j] = pltpu.async_copy(rows.at[j % 2], acc.at[didx.at[j]],
                                     sems, add=True)
            if j >= 1:
                sd[j - 1].wait()
            if j + 1 < _GRP:
                gd[j + 1] = pltpu.async_copy(table.at[sidx.at[j + 1]],
                                             rows.at[(j + 1) % 2], semg)
        sd[_GRP - 1].wait()
        return carry

    lax.fori_loop(0, _NGRP, group, 0)

    plsc.subcore_barrier()
    pltpu.sync_copy(acc.at[pl.ds(nb, _WB), :], out_hbm.at[c, pl.ds(nb, _WB), :])


@functools.lru_cache(maxsize=None)
def _make_sc_kernels():
    mesh = plsc.VectorSubcoreMesh(
        core_axis_name="c", subcore_axis_name="s",
        num_cores=_NC, num_subcores=_NS)
    params = pltpu.CompilerParams(use_tc_tiling_on_sc=False)
    deg = pl.kernel(
        _deg_body,
        out_type=jax.ShapeDtypeStruct((_NC, _NPAD), jnp.float32),
        mesh=mesh,
        compiler_params=params,
        scratch_types=[
            pltpu.VMEM_SHARED((_NPAD,), jnp.float32),
            pltpu.VMEM((_GRP, _C), jnp.int32),
            pltpu.VMEM((_C,), jnp.float32),
            pltpu.VMEM((2048,), jnp.float32),
        ],
    )
    agg = pl.kernel(
        _agg_body,
        out_type=jax.ShapeDtypeStruct((_NC, _NPAD, _DH), jnp.float32),
        mesh=mesh,
        compiler_params=params,
        scratch_types=[
            pltpu.VMEM_SHARED((_NPAD, _DH), jnp.float32),
            pltpu.VMEM_SHARED((_NPAD, _DH), jnp.float32),
            pltpu.VMEM((_GRP, _C), jnp.int32),
            pltpu.VMEM((_GRP, _C), jnp.int32),
            pltpu.VMEM((2, _C, _DH), jnp.float32),
            pltpu.SemaphoreType.DMA,
            pltpu.SemaphoreType.DMA,
        ],
    )
    return deg, agg


def _tc1_body(x_ref, p0_ref, p1_ref, w0_ref, b0_ref, w1_ref, b1_ref,
              wg0_ref, outl_ref, outr_ref):
    x = x_ref[...]
    h = jnp.maximum(
        jnp.dot(x, w0_ref[...], preferred_element_type=jnp.float32)
        + b0_ref[...], 0.0)
    h = jnp.maximum(
        jnp.dot(h, w1_ref[...], preferred_element_type=jnp.float32)
        + b1_ref[...], 0.0)
    y = jnp.dot(h, wg0_ref[...], preferred_element_type=jnp.float32)
    dinv = 1.0 / jnp.sqrt(p0_ref[...] + p1_ref[...] + 1.0)
    hs = y * dinv
    outl_ref[...] = hs[:, :_DH]
    outr_ref[...] = hs[:, _DH:]


def _tc2_body(al_ref, ar_ref, p0_ref, p1_ref, bg_ref, g_ref, bl_ref,
              wg1_ref, outl_ref, outr_ref):
    dinv = 1.0 / jnp.sqrt(p0_ref[...] + p1_ref[...] + 1.0)
    a = jnp.concatenate([al_ref[...], ar_ref[...]], axis=-1)
    t = a * dinv + bg_ref[...]
    t = jnp.maximum(t, 0.0)
    mu = jnp.mean(t, axis=-1, keepdims=True)
    d = t - mu
    var = jnp.mean(d * d, axis=-1, keepdims=True)
    t = d / jnp.sqrt(var + 1e-5) * g_ref[...] + bl_ref[...]
    y = jnp.dot(t, wg1_ref[...], preferred_element_type=jnp.float32)
    hs = y * dinv
    outl_ref[...] = hs[:, :_DH]
    outr_ref[...] = hs[:, _DH:]


def _tc3_body(al_ref, ar_ref, p0_ref, p1_ref, bg_ref, g_ref, bl_ref,
              wp_ref, bp_ref, wo_ref, bo_ref, out_ref):
    dinv = 1.0 / jnp.sqrt(p0_ref[...] + p1_ref[...] + 1.0)
    a = jnp.concatenate([al_ref[...], ar_ref[...]], axis=-1)
    t = a * dinv + bg_ref[...]
    t = jnp.maximum(t, 0.0)
    mu = jnp.mean(t, axis=-1, keepdims=True)
    d = t - mu
    var = jnp.mean(d * d, axis=-1, keepdims=True)
    t = d / jnp.sqrt(var + 1e-5) * g_ref[...] + bl_ref[...]
    h = jnp.maximum(
        jnp.dot(t, wp_ref[...], preferred_element_type=jnp.float32)
        + bp_ref[...], 0.0)
    out_ref[...] = (
        jnp.dot(h, wo_ref[...], preferred_element_type=jnp.float32)
        + bo_ref[...])


def _row_spec(width):
    return pl.BlockSpec((_BN, width), lambda i: (i, 0))


def _full_spec(r, c):
    return pl.BlockSpec((r, c), lambda i: (0, 0))


_GRID = (_N // _BN,)

_half_out = [jax.ShapeDtypeStruct((_NPAD, _DH), jnp.float32)] * 2
_half_out_specs = [_row_spec(_DH), _row_spec(_DH)]

_tc1_call = pl.pallas_call(
    _tc1_body,
    grid=_GRID,
    in_specs=[_row_spec(_D), _row_spec(1), _row_spec(1),
              _full_spec(_D, _D), _full_spec(1, _D),
              _full_spec(_D, _D), _full_spec(1, _D),
              _full_spec(_D, _D)],
    out_specs=_half_out_specs,
    out_shape=_half_out,
)

_tc2_call = pl.pallas_call(
    _tc2_body,
    grid=_GRID,
    in_specs=[_row_spec(_DH), _row_spec(_DH),
              _row_spec(1), _row_spec(1),
              _full_spec(1, _D), _full_spec(1, _D), _full_spec(1, _D),
              _full_spec(_D, _D)],
    out_specs=_half_out_specs,
    out_shape=_half_out,
)

_tc3_call = pl.pallas_call(
    _tc3_body,
    grid=_GRID,
    in_specs=[_row_spec(_DH), _row_spec(_DH),
              _row_spec(1), _row_spec(1),
              _full_spec(1, _D), _full_spec(1, _D), _full_spec(1, _D),
              _full_spec(_D, _D), _full_spec(1, _D),
              _full_spec(_D, 1), _full_spec(1, 1)],
    out_specs=pl.BlockSpec((_BN, 1), lambda i: (i, 0)),
    out_shape=jax.ShapeDtypeStruct((_N, 1), jnp.float32),
)


def kernel(x, edge_index, W_pre0, b_pre0, W_pre1, b_pre1, W_g0, b_g0,
           W_g1, b_g1, ln0_g, ln0_b, ln1_g, ln1_b, W_post, b_post,
           W_out, b_out):
    npad_e = _EP - _E
    pad_src = jnp.zeros((npad_e,), jnp.int32)
    pad_dst = _N + (jnp.arange(npad_e, dtype=jnp.int32) % (_NPAD - _N))
    src2 = jnp.concatenate([edge_index[0], pad_src]).reshape(_ROWS, _C)
    dst2 = jnp.concatenate([edge_index[1], pad_dst]).reshape(_ROWS, _C)
    b_pre0r = b_pre0.reshape(1, _D)
    b_pre1r = b_pre1.reshape(1, _D)
    b_g0r = b_g0.reshape(1, _D)
    b_g1r = b_g1.reshape(1, _D)
    ln0_gr = ln0_g.reshape(1, _D)
    ln0_br = ln0_b.reshape(1, _D)
    ln1_gr = ln1_g.reshape(1, _D)
    ln1_br = ln1_b.reshape(1, _D)
    b_postr = b_post.reshape(1, _D)
    b_outr = b_out.reshape(1, 1)

    _deg_call, _agg_call = _make_sc_kernels()

    degp = _deg_call(dst2)
    p0 = degp[0, :_N].reshape(_N, 1)
    p1 = degp[1, :_N].reshape(_N, 1)

    hs1l, hs1r = _tc1_call(x, p0, p1, W_pre0, b_pre0r, W_pre1, b_pre1r, W_g0)
    aggp1 = _agg_call(hs1l, hs1r, src2, dst2)
    hs2l, hs2r = _tc2_call(aggp1[0, :_N], aggp1[1, :_N], p0, p1, b_g0r,
                           ln0_gr, ln0_br, W_g1)
    aggp2 = _agg_call(hs2l, hs2r, src2, dst2)
    out = _tc3_call(aggp2[0, :_N], aggp2[1, :_N], p0, p1, b_g1r,
                    ln1_gr, ln1_br, W_post, b_postr, W_out, b_outr)
    return out


# speedup vs baseline: 1.0906x; 1.0906x over previous
import functools

import jax
import jax.numpy as jnp
from jax import lax
from jax.experimental import pallas as pl
from jax.experimental.pallas import tpu as pltpu
from jax.experimental.pallas import tpu_sc as plsc

_N = 10000
_E = 320000
_D = 128
_DH = _D // 2
_NC = 2
_NS = 16
_NW = _NC * _NS
_C = 128
_EP = 327680
_ROWS = _EP // _C
_RPT = _ROWS // _NS
_GRP = 16
_NGRP = _RPT // _GRP
_NPAD = 10240
_WB = _NPAD // _NS
_BN = 1000


def _deg_body(dst_hbm, out_hbm, hist, idx, ones, zbuf):
    c = lax.axis_index("c")
    s = lax.axis_index("s")
    wid = s * _NC + c

    def fill_ones(i, carry):
        ones[pl.ds(i * 16, 16)] = jnp.ones((16,), jnp.float32)
        return carry

    lax.fori_loop(0, _C // 16, fill_ones, 0)

    @pl.when(s == 0)
    def _init():
        def fill_z(i, carry):
            zbuf[pl.ds(i * 16, 16)] = jnp.zeros((16,), jnp.float32)
            return carry

        lax.fori_loop(0, 128, fill_z, 0)
        for j in range(_NPAD // 2048):
            pltpu.sync_copy(zbuf, hist.at[pl.ds(j * 2048, 2048)])

    plsc.subcore_barrier()

    base_row = wid * (_ROWS // _NW)

    def group(g, carry):
        pltpu.sync_copy(dst_hbm.at[pl.ds(base_row + g * _GRP, _GRP), :], idx)
        for j in range(_GRP):
            pltpu.sync_copy(ones, hist.at[idx.at[j]], add=True)
        return carry

    lax.fori_loop(0, (_ROWS // _NW) // _GRP, group, 0)

    plsc.subcore_barrier()
    nb = s * _WB
    pltpu.sync_copy(hist.at[pl.ds(nb, _WB)], out_hbm.at[c, pl.ds(nb, _WB)])


def _agg_body(hsL_hbm, hsR_hbm, src_hbm, dst_hbm, out_hbm,
              table, acc, sidx, didx, rows, semg, sems):
    c = lax.axis_index("c")
    s = lax.axis_index("s")
    nb = s * _WB

    @pl.when(c == 0)
    def _stage0():
        pltpu.sync_copy(hsL_hbm.at[pl.ds(nb, _WB), :], table.at[pl.ds(nb, _WB), :])
        pltpu.sync_copy(hsL_hbm.at[pl.ds(nb, _WB), :], acc.at[pl.ds(nb, _WB), :])

    @pl.when(c == 1)
    def _stage1():
        pltpu.sync_copy(hsR_hbm.at[pl.ds(nb, _WB), :], table.at[pl.ds(nb, _WB), :])
        pltpu.sync_copy(hsR_hbm.at[pl.ds(nb, _WB), :], acc.at[pl.ds(nb, _WB), :])

    plsc.subcore_barrier()

    base_row = s * _RPT

    def group(g, carry):
        pltpu.sync_copy(src_hbm.at[pl.ds(base_row + g * _GRP, _GRP), :], sidx)
        pltpu.sync_copy(dst_hbm.at[pl.ds(base_row + g * _GRP, _GRP), :], didx)
        gd = [None] * _GRP
        sd = [None] * _GRP
        for k in range(3):
            gd[k] = pltpu.async_copy(table.at[sidx.at[k]], rows.at[k], semg)
        for j in range(_GRP):
            gd[j].wait()
            sd[j] = pltpu.async_copy(rows.at[j % 4], acc.at[didx.at[j]],
                                     sems, add=True)
            if j >= 1:
                sd[j - 1].wait()
            if j + 3 < _GRP:
                gd[j + 3] = pltpu.async_copy(table.at[sidx.at[j + 3]],
                                             rows.at[(j + 3) % 4], semg)
        sd[_GRP - 1].wait()
        return carry

    lax.fori_loop(0, _NGRP, group, 0)

    plsc.subcore_barrier()
    pltpu.sync_copy(acc.at[pl.ds(nb, _WB), :], out_hbm.at[c, pl.ds(nb, _WB), :])


@functools.lru_cache(maxsize=None)
def _make_sc_kernels():
    mesh = plsc.VectorSubcoreMesh(
        core_axis_name="c", subcore_axis_name="s",
        num_cores=_NC, num_subcores=_NS)
    params = pltpu.CompilerParams(use_tc_tiling_on_sc=False)
    deg = pl.kernel(
        _deg_body,
        out_type=jax.ShapeDtypeStruct((_NC, _NPAD), jnp.float32),
        mesh=mesh,
        compiler_params=params,
        scratch_types=[
            pltpu.VMEM_SHARED((_NPAD,), jnp.float32),
            pltpu.VMEM((_GRP, _C), jnp.int32),
            pltpu.VMEM((_C,), jnp.float32),
            pltpu.VMEM((2048,), jnp.float32),
        ],
    )
    agg = pl.kernel(
        _agg_body,
        out_type=jax.ShapeDtypeStruct((_NC, _NPAD, _DH), jnp.float32),
        mesh=mesh,
        compiler_params=params,
        scratch_types=[
            pltpu.VMEM_SHARED((_NPAD, _DH), jnp.float32),
            pltpu.VMEM_SHARED((_NPAD, _DH), jnp.float32),
            pltpu.VMEM((_GRP, _C), jnp.int32),
            pltpu.VMEM((_GRP, _C), jnp.int32),
            pltpu.VMEM((4, _C, _DH), jnp.float32),
            pltpu.SemaphoreType.DMA,
            pltpu.SemaphoreType.DMA,
        ],
    )
    return deg, agg


def _tc1_body(x_ref, p0_ref, p1_ref, w0_ref, b0_ref, w1_ref, b1_ref,
              wg0_ref, outl_ref, outr_ref):
    x = x_ref[...]
    h = jnp.maximum(
        jnp.dot(x, w0_ref[...], preferred_element_type=jnp.float32)
        + b0_ref[...], 0.0)
    h = jnp.maximum(
        jnp.dot(h, w1_ref[...], preferred_element_type=jnp.float32)
        + b1_ref[...], 0.0)
    y = jnp.dot(h, wg0_ref[...], preferred_element_type=jnp.float32)
    dinv = 1.0 / jnp.sqrt(p0_ref[...] + p1_ref[...] + 1.0)
    hs = y * dinv
    outl_ref[...] = hs[:, :_DH]
    outr_ref[...] = hs[:, _DH:]


def _tc2_body(al_ref, ar_ref, p0_ref, p1_ref, bg_ref, g_ref, bl_ref,
              wg1_ref, outl_ref, outr_ref):
    dinv = 1.0 / jnp.sqrt(p0_ref[...] + p1_ref[...] + 1.0)
    a = jnp.concatenate([al_ref[...], ar_ref[...]], axis=-1)
    t = a * dinv + bg_ref[...]
    t = jnp.maximum(t, 0.0)
    mu = jnp.mean(t, axis=-1, keepdims=True)
    d = t - mu
    var = jnp.mean(d * d, axis=-1, keepdims=True)
    t = d / jnp.sqrt(var + 1e-5) * g_ref[...] + bl_ref[...]
    y = jnp.dot(t, wg1_ref[...], preferred_element_type=jnp.float32)
    hs = y * dinv
    outl_ref[...] = hs[:, :_DH]
    outr_ref[...] = hs[:, _DH:]


def _tc3_body(al_ref, ar_ref, p0_ref, p1_ref, bg_ref, g_ref, bl_ref,
              wp_ref, bp_ref, wo_ref, bo_ref, out_ref):
    dinv = 1.0 / jnp.sqrt(p0_ref[...] + p1_ref[...] + 1.0)
    a = jnp.concatenate([al_ref[...], ar_ref[...]], axis=-1)
    t = a * dinv + bg_ref[...]
    t = jnp.maximum(t, 0.0)
    mu = jnp.mean(t, axis=-1, keepdims=True)
    d = t - mu
    var = jnp.mean(d * d, axis=-1, keepdims=True)
    t = d / jnp.sqrt(var + 1e-5) * g_ref[...] + bl_ref[...]
    h = jnp.maximum(
        jnp.dot(t, wp_ref[...], preferred_element_type=jnp.float32)
        + bp_ref[...], 0.0)
    out_ref[...] = (
        jnp.dot(h, wo_ref[...], preferred_element_type=jnp.float32)
        + bo_ref[...])


def _row_spec(width):
    return pl.BlockSpec((_BN, width), lambda i: (i, 0))


def _full_spec(r, c):
    return pl.BlockSpec((r, c), lambda i: (0, 0))


_GRID = (_N // _BN,)

_half_out = [jax.ShapeDtypeStruct((_NPAD, _DH), jnp.float32)] * 2
_half_out_specs = [_row_spec(_DH), _row_spec(_DH)]

_tc1_call = pl.pallas_call(
    _tc1_body,
    grid=_GRID,
    in_specs=[_row_spec(_D), _row_spec(1), _row_spec(1),
              _full_spec(_D, _D), _full_spec(1, _D),
              _full_spec(_D, _D), _full_spec(1, _D),
              _full_spec(_D, _D)],
    out_specs=_half_out_specs,
    out_shape=_half_out,
)

_tc2_call = pl.pallas_call(
    _tc2_body,
    grid=_GRID,
    in_specs=[_row_spec(_DH), _row_spec(_DH),
              _row_spec(1), _row_spec(1),
              _full_spec(1, _D), _full_spec(1, _D), _full_spec(1, _D),
              _full_spec(_D, _D)],
    out_specs=_half_out_specs,
    out_shape=_half_out,
)

_tc3_call = pl.pallas_call(
    _tc3_body,
    grid=_GRID,
    in_specs=[_row_spec(_DH), _row_spec(_DH),
              _row_spec(1), _row_spec(1),
              _full_spec(1, _D), _full_spec(1, _D), _full_spec(1, _D),
              _full_spec(_D, _D), _full_spec(1, _D),
              _full_spec(_D, 1), _full_spec(1, 1)],
    out_specs=pl.BlockSpec((_BN, 1), lambda i: (i, 0)),
    out_shape=jax.ShapeDtypeStruct((_N, 1), jnp.float32),
)


def kernel(x, edge_index, W_pre0, b_pre0, W_pre1, b_pre1, W_g0, b_g0,
           W_g1, b_g1, ln0_g, ln0_b, ln1_g, ln1_b, W_post, b_post,
           W_out, b_out):
    npad_e = _EP - _E
    pad_src = jnp.zeros((npad_e,), jnp.int32)
    pad_dst = _N + (jnp.arange(npad_e, dtype=jnp.int32) % (_NPAD - _N))
    src2 = jnp.concatenate([edge_index[0], pad_src]).reshape(_ROWS, _C)
    dst2 = jnp.concatenate([edge_index[1], pad_dst]).reshape(_ROWS, _C)
    b_pre0r = b_pre0.reshape(1, _D)
    b_pre1r = b_pre1.reshape(1, _D)
    b_g0r = b_g0.reshape(1, _D)
    b_g1r = b_g1.reshape(1, _D)
    ln0_gr = ln0_g.reshape(1, _D)
    ln0_br = ln0_b.reshape(1, _D)
    ln1_gr = ln1_g.reshape(1, _D)
    ln1_br = ln1_b.reshape(1, _D)
    b_postr = b_post.reshape(1, _D)
    b_outr = b_out.reshape(1, 1)

    _deg_call, _agg_call = _make_sc_kernels()

    degp = _deg_call(dst2)
    p0 = degp[0, :_N].reshape(_N, 1)
    p1 = degp[1, :_N].reshape(_N, 1)

    hs1l, hs1r = _tc1_call(x, p0, p1, W_pre0, b_pre0r, W_pre1, b_pre1r, W_g0)
    aggp1 = _agg_call(hs1l, hs1r, src2, dst2)
    hs2l, hs2r = _tc2_call(aggp1[0, :_N], aggp1[1, :_N], p0, p1, b_g0r,
                           ln0_gr, ln0_br, W_g1)
    aggp2 = _agg_call(hs2l, hs2r, src2, dst2)
    out = _tc3_call(aggp2[0, :_N], aggp2[1, :_N], p0, p1, b_g1r,
                    ln1_gr, ln1_br, W_post, b_postr, W_out, b_outr)
    return out
